# baseline (speedup 1.0000x reference)
import functools

import jax
import jax.numpy as jnp
from jax import lax
from jax.experimental import pallas as pl
from jax.experimental.pallas import tpu as pltpu
from jax.experimental.pallas import tpu_sc as plsc

LANES = 16
N_XCHUNKS = 4
V_TILE = 128


def _make_sc_kernel(V, EMB, B, L, n_workers):
    sents_per_worker = B // n_workers
    n_blocks = sents_per_worker // LANES
    V_pad = ((V + V_TILE - 1) // V_TILE) * V_TILE
    n_vtiles = V_pad // V_TILE
    mesh = plsc.VectorSubcoreMesh(core_axis_name="c", subcore_axis_name="s")

    @functools.partial(
        pl.kernel,
        mesh=mesh,
        out_type=jax.ShapeDtypeStruct((B,), jnp.float32),
        scratch_types=[
            pltpu.VMEM((L, sents_per_worker), jnp.int32),
            pltpu.VMEM((V_pad,), jnp.float32),
            pltpu.VMEM((sents_per_worker,), jnp.float32),
            pltpu.VMEM((EMB, V_TILE), jnp.float32),
            pltpu.VMEM((EMB, LANES), jnp.float32),
            pltpu.VMEM((LANES,), jnp.float32),
            pltpu.VMEM_SHARED((V_pad,), jnp.float32),
            [pltpu.SemaphoreType.DMA] * N_XCHUNKS,
        ],
        compiler_params=pltpu.CompilerParams(needs_layout_passes=False),
    )
    def sc_kernel(xt_hbm, tableT_hbm, w_rep_hbm, b_hbm, out_hbm,
                  idx_v, tw_v, out_v, tbl_v, w_v, b_v, tw_sh, sems):
        n_cores = 2
        sid = lax.axis_index("s")
        wid = sid * n_cores + lax.axis_index("c")
        base_s = wid * sents_per_worker
        chunk = sents_per_worker // N_XCHUNKS

        copies = [
            pltpu.async_copy(
                xt_hbm.at[:, pl.ds(base_s + c * chunk, chunk)],
                idx_v.at[:, pl.ds(c * chunk, chunk)],
                sems[c])
            for c in range(N_XCHUNKS)
        ]

        pltpu.sync_copy(w_rep_hbm, w_v)
        pltpu.sync_copy(b_hbm, b_v)
        v0 = jnp.minimum(sid, n_vtiles - 1) * V_TILE
        inv_l = 1.0 / L
        n_groups = V_TILE // LANES
        zero16 = jnp.zeros((LANES,), jnp.float32)

        @pl.when(sid < n_vtiles)
        def _compute_tw():
            pltpu.sync_copy(tableT_hbm.at[:, pl.ds(v0, V_TILE)], tbl_v)

            def e_body(e, accs):
                wv = w_v[e, :]
                return tuple(
                    accs[g] + wv * tbl_v[e, pl.ds(g * LANES, LANES)]
                    for g in range(n_groups))

            accws = lax.fori_loop(0, EMB, e_body, (zero16,) * n_groups,
                                  unroll=4)
            for g in range(n_groups):
                tw_v[pl.ds(g * LANES, LANES)] = accws[g] * inv_l

            pltpu.sync_copy(tw_v.at[pl.ds(0, V_TILE)],
                            tw_sh.at[pl.ds(v0, V_TILE)])

        plsc.subcore_barrier()
        pltpu.sync_copy(tw_sh, tw_v)

        bvec = b_v[...]

        for c in range(N_XCHUNKS):
            copies[c].wait()

            def blk_body(blk, _):
                s0 = blk * LANES

                def body(t, acc):
                    xv = idx_v[t, pl.ds(s0, LANES)]
                    tv = plsc.load_gather(tw_v, [xv])
                    return acc + tv

                acc = lax.fori_loop(0, L, body,
                                    jnp.zeros((LANES,), jnp.float32),
                                    unroll=8)
                pred = 1.0 / (1.0 + jnp.exp(-(acc + bvec)))
                out_v[pl.ds(s0, LANES)] = pred
                return 0

            lax.fori_loop(c * (n_blocks // N_XCHUNKS),
                          (c + 1) * (n_blocks // N_XCHUNKS), blk_body, 0)

        pltpu.sync_copy(out_v, out_hbm.at[pl.ds(base_s, sents_per_worker)])

    return sc_kernel


def kernel(x, table, W, b):
    B, L = x.shape
    V, EMB = table.shape

    b_rep = jnp.broadcast_to(b, (LANES,))
    w_rep = jnp.broadcast_to(W, (EMB, LANES))
    V_pad = ((V + V_TILE - 1) // V_TILE) * V_TILE
    tableT_p = jnp.pad(table.T, ((0, 0), (0, V_pad - V)))
    out = _make_sc_kernel(V, EMB, B, L, 32)(
        x.T.astype(jnp.int32), tableT_p, w_rep, b_rep)
    return out.reshape(B, 1)


# speedup vs baseline: 1.1682x; 1.1682x over previous
import functools

import jax
import jax.numpy as jnp
from jax import lax
from jax.experimental import pallas as pl
from jax.experimental.pallas import tpu as pltpu
from jax.experimental.pallas import tpu_sc as plsc

LANES = 16
N_XCHUNKS = 4


def _tw_tc_kernel(tableT_ref, wT_ref, b_ref, out_ref, *, inv_l):
    tT = tableT_ref[...]
    wT = wT_ref[...]
    tw = jnp.dot(wT, tT, preferred_element_type=jnp.float32)
    out_ref[...] = (tw + b_ref[0]) * inv_l


def _make_sc_lookup(V, B, L, n_workers):
    sents_per_worker = B // n_workers
    n_blocks = sents_per_worker // LANES
    mesh = plsc.VectorSubcoreMesh(core_axis_name="c", subcore_axis_name="s")

    @functools.partial(
        pl.kernel,
        mesh=mesh,
        out_type=jax.ShapeDtypeStruct((B,), jnp.float32),
        scratch_types=[
            pltpu.VMEM((L, sents_per_worker), jnp.int32),
            pltpu.VMEM((V,), jnp.float32),
            pltpu.VMEM((sents_per_worker,), jnp.float32),
            [pltpu.SemaphoreType.DMA] * N_XCHUNKS,
        ],
        compiler_params=pltpu.CompilerParams(needs_layout_passes=False),
    )
    def sc_lookup(xt_hbm, tw_hbm, out_hbm, idx_v, tw_v, out_v, sems):
        n_cores = 2
        wid = lax.axis_index("s") * n_cores + lax.axis_index("c")
        base_s = wid * sents_per_worker
        chunk = sents_per_worker // N_XCHUNKS

        copies = [
            pltpu.async_copy(
                xt_hbm.at[:, pl.ds(base_s + c * chunk, chunk)],
                idx_v.at[:, pl.ds(c * chunk, chunk)],
                sems[c])
            for c in range(N_XCHUNKS)
        ]
        pltpu.sync_copy(tw_hbm.at[0], tw_v)

        for c in range(N_XCHUNKS):
            copies[c].wait()

            def blk_body(blk, _):
                s0 = blk * LANES

                def body(t, acc):
                    xv = idx_v[t, pl.ds(s0, LANES)]
                    tv = plsc.load_gather(tw_v, [xv])
                    return acc + tv

                acc = lax.fori_loop(0, L, body,
                                    jnp.zeros((LANES,), jnp.float32),
                                    unroll=8)
                pred = 1.0 / (1.0 + jnp.exp(-acc))
                out_v[pl.ds(s0, LANES)] = pred
                return 0

            lax.fori_loop(c * (n_blocks // N_XCHUNKS),
                          (c + 1) * (n_blocks // N_XCHUNKS), blk_body, 0)

        pltpu.sync_copy(out_v, out_hbm.at[pl.ds(base_s, sents_per_worker)])

    return sc_lookup


def kernel(x, table, W, b):
    B, L = x.shape
    V, EMB = table.shape

    tw = pl.pallas_call(
        functools.partial(_tw_tc_kernel, inv_l=1.0 / L),
        out_shape=jax.ShapeDtypeStruct((1, V), jnp.float32),
    )(table.T, W.T, b)

    out = _make_sc_lookup(V, B, L, 32)(x.T.astype(jnp.int32), tw)
    return out.reshape(B, 1)


# speedup vs baseline: 1.1844x; 1.0138x over previous
import functools

import jax
import jax.numpy as jnp
from jax import lax
from jax.experimental import pallas as pl
from jax.experimental.pallas import tpu as pltpu
from jax.experimental.pallas import tpu_sc as plsc

LANES = 16
N_XCHUNKS = 4


def _tw_tc_kernel(tableT_ref, wT_ref, b_ref, out_ref, *, inv_l):
    tT = tableT_ref[...]
    wT = wT_ref[...]
    tw = jnp.dot(wT, tT, preferred_element_type=jnp.float32)
    out_ref[...] = (tw + b_ref[0]) * inv_l


def _make_sc_lookup(V, B, L, n_workers):
    sents_per_worker = B // n_workers
    n_blocks = sents_per_worker // LANES
    mesh = plsc.VectorSubcoreMesh(core_axis_name="c", subcore_axis_name="s")

    @functools.partial(
        pl.kernel,
        mesh=mesh,
        out_type=jax.ShapeDtypeStruct((B,), jnp.float32),
        scratch_types=[
            pltpu.VMEM((L, sents_per_worker), jnp.int32),
            pltpu.VMEM((V,), jnp.float32),
            pltpu.VMEM((sents_per_worker,), jnp.float32),
            [pltpu.SemaphoreType.DMA] * N_XCHUNKS,
        ],
        compiler_params=pltpu.CompilerParams(needs_layout_passes=False),
    )
    def sc_lookup(xt_hbm, tw_hbm, out_hbm, idx_v, tw_v, out_v, sems):
        n_cores = 2
        wid = lax.axis_index("s") * n_cores + lax.axis_index("c")
        base_s = wid * sents_per_worker
        chunk = sents_per_worker // N_XCHUNKS

        copies = [
            pltpu.async_copy(
                xt_hbm.at[:, pl.ds(base_s + c * chunk, chunk)],
                idx_v.at[:, pl.ds(c * chunk, chunk)],
                sems[c])
            for c in range(N_XCHUNKS)
        ]
        pltpu.sync_copy(tw_hbm.at[0], tw_v)

        for c in range(N_XCHUNKS):
            copies[c].wait()

            def blk_body(blk, _):
                s0 = blk * (2 * LANES)

                def body(t, accs):
                    a0, a1 = accs
                    xv0 = idx_v[t, pl.ds(s0, LANES)]
                    xv1 = idx_v[t, pl.ds(s0 + LANES, LANES)]
                    tv0 = plsc.load_gather(tw_v, [xv0])
                    tv1 = plsc.load_gather(tw_v, [xv1])
                    return a0 + tv0, a1 + tv1

                zero = jnp.zeros((LANES,), jnp.float32)
                a0, a1 = lax.fori_loop(0, L, body, (zero, zero),
                                       unroll=8)
                out_v[pl.ds(s0, LANES)] = 1.0 / (1.0 + jnp.exp(-a0))
                out_v[pl.ds(s0 + LANES, LANES)] = 1.0 / (1.0 + jnp.exp(-a1))
                return 0

            half_blocks = n_blocks // 2
            lax.fori_loop(c * (half_blocks // N_XCHUNKS),
                          (c + 1) * (half_blocks // N_XCHUNKS), blk_body, 0)

        pltpu.sync_copy(out_v, out_hbm.at[pl.ds(base_s, sents_per_worker)])

    return sc_lookup


def kernel(x, table, W, b):
    B, L = x.shape
    V, EMB = table.shape

    tw = pl.pallas_call(
        functools.partial(_tw_tc_kernel, inv_l=1.0 / L),
        out_shape=jax.ShapeDtypeStruct((1, V), jnp.float32),
    )(table.T, W.T, b)

    out = _make_sc_lookup(V, B, L, 32)(x.T.astype(jnp.int32), tw)
    return out.reshape(B, 1)
